# baseline (speedup 1.0000x reference)
import functools

import jax
import jax.numpy as jnp
from jax import lax
from jax.experimental import pallas as pl
from jax.experimental.pallas import tpu as pltpu
from jax.experimental.pallas import tpu_sc as plsc

B = 16384
D = 128
NC = 2
NS = 16
NW = NC * NS
BPW = B // NW
CH = 64
NCHUNK = BPW // CH
RUNROLL = 4

_mesh = plsc.VectorSubcoreMesh(core_axis_name="c", subcore_axis_name="s")


@functools.partial(
    pl.kernel,
    mesh=_mesh,
    compiler_params=pltpu.CompilerParams(needs_layout_passes=False),
    out_type=jax.ShapeDtypeStruct((B,), jnp.float32),
    scratch_types=[
        pltpu.VMEM((BPW,), jnp.int32),
        [pltpu.VMEM((CH, D), jnp.float32) for _ in range(2)],
        [pltpu.VMEM((CH, D), jnp.float32) for _ in range(2)],
        pltpu.VMEM((BPW,), jnp.float32),
        [pltpu.SemaphoreType.DMA for _ in range(2)],
        [pltpu.SemaphoreType.DMA for _ in range(2)],
        pltpu.SemaphoreType.DMA,
        pltpu.SemaphoreType.DMA,
    ],
)
def _sim_kernel(x_hbm, idx_hbm, out_hbm, idx_v, own_v, gth_v, out_v, gsem, osem, isem, wsem):
    wid = lax.axis_index("s") * NC + lax.axis_index("c")
    stripe = wid * BPW
    lanes = lax.iota(jnp.int32, 16)
    last_lane = lanes == 15

    h_idx = pltpu.async_copy(idx_hbm.at[pl.ds(stripe, BPW)], idx_v, isem)

    def issue(ci, bi):
        pltpu.async_copy(
            x_hbm.at[pl.ds(stripe + ci * CH, CH)], own_v[bi], osem[bi]
        )
        pltpu.async_copy(
            x_hbm.at[idx_v.at[pl.ds(ci * CH, CH)]], gth_v[bi], gsem[bi]
        )

    def compute_row(r, r_out, ov, gv):
        acc0 = jnp.abs(ov[r, pl.ds(0, 16)] - gv[r, pl.ds(0, 16)])
        acc1 = jnp.abs(ov[r, pl.ds(16, 16)] - gv[r, pl.ds(16, 16)])
        for k in range(2, D // 16, 2):
            acc0 = acc0 + jnp.abs(ov[r, pl.ds(k * 16, 16)] - gv[r, pl.ds(k * 16, 16)])
            acc1 = acc1 + jnp.abs(ov[r, pl.ds(k * 16 + 16, 16)] - gv[r, pl.ds(k * 16 + 16, 16)])
        cs = plsc.cumsum(acc0 + acc1)
        plsc.store_scatter(
            out_v, [jnp.full((16,), r_out, jnp.int32)], jnp.exp(-cs), mask=last_lane
        )

    pltpu.async_copy(x_hbm.at[pl.ds(stripe, CH)], own_v[0], osem[0])
    h_idx.wait()
    pltpu.async_copy(x_hbm.at[idx_v.at[pl.ds(0, CH)]], gth_v[0], gsem[0])
    issue(1, 1)

    def superchunk(s, carry):
        for bi in range(2):
            ci = s * 2 + bi
            off = ci * CH
            pltpu.make_async_copy(x_hbm.at[pl.ds(0, CH)], gth_v[bi], gsem[bi]).wait()
            pltpu.make_async_copy(x_hbm.at[pl.ds(0, CH)], own_v[bi], osem[bi]).wait()

            ov, gv = own_v[bi], gth_v[bi]

            @plsc.parallel_loop(0, CH, step=1, unroll=RUNROLL)
            def _rows(r):
                compute_row(r, off + r, ov, gv)

            pltpu.async_copy(
                out_v.at[pl.ds(off, CH)], out_hbm.at[pl.ds(stripe + off, CH)], wsem
            )

            @pl.when(ci + 2 < NCHUNK)
            def _prefetch():
                issue(ci + 2, bi)

        return carry

    lax.fori_loop(0, NCHUNK // 2, superchunk, 0)

    pltpu.make_async_copy(out_v, out_hbm.at[pl.ds(stripe, BPW)], wsem).wait()


def kernel(x, idx):
    return _sim_kernel(x, idx).reshape(B, 1)


# speedup vs baseline: 1.0222x; 1.0222x over previous
import functools

import jax
import jax.numpy as jnp
from jax import lax
from jax.experimental import pallas as pl
from jax.experimental.pallas import tpu as pltpu
from jax.experimental.pallas import tpu_sc as plsc

B = 16384
D = 128
NC = 2
NS = 16
NW = NC * NS
BPW = B // NW
CH = 64
NCHUNK = BPW // CH
NBUF = 4
RUNROLL = 4

_mesh = plsc.VectorSubcoreMesh(core_axis_name="c", subcore_axis_name="s")


@functools.partial(
    pl.kernel,
    mesh=_mesh,
    compiler_params=pltpu.CompilerParams(needs_layout_passes=False),
    out_type=jax.ShapeDtypeStruct((B,), jnp.float32),
    scratch_types=[
        pltpu.VMEM((BPW,), jnp.int32),
        [pltpu.VMEM((CH, D), jnp.float32) for _ in range(NBUF)],
        [pltpu.VMEM((CH, D), jnp.float32) for _ in range(NBUF)],
        pltpu.VMEM((BPW,), jnp.float32),
        [pltpu.SemaphoreType.DMA for _ in range(NBUF)],
        [pltpu.SemaphoreType.DMA for _ in range(NBUF)],
        pltpu.SemaphoreType.DMA,
        pltpu.SemaphoreType.DMA,
    ],
)
def _sim_kernel(x_hbm, idx_hbm, out_hbm, idx_v, own_v, gth_v, out_v, gsem, osem, isem, wsem):
    wid = lax.axis_index("s") * NC + lax.axis_index("c")
    stripe = wid * BPW
    lanes = lax.iota(jnp.int32, 16)
    last_lane = lanes == 15

    h_idx = pltpu.async_copy(idx_hbm.at[pl.ds(stripe, BPW)], idx_v, isem)

    def issue(ci, bi):
        pltpu.async_copy(
            x_hbm.at[pl.ds(stripe + ci * CH, CH)], own_v[bi], osem[bi]
        )
        pltpu.async_copy(
            x_hbm.at[idx_v.at[pl.ds(ci * CH, CH)]], gth_v[bi], gsem[bi]
        )

    def compute_row(r, r_out, ov, gv):
        acc0 = jnp.abs(ov[r, pl.ds(0, 16)] - gv[r, pl.ds(0, 16)])
        acc1 = jnp.abs(ov[r, pl.ds(16, 16)] - gv[r, pl.ds(16, 16)])
        for k in range(2, D // 16, 2):
            acc0 = acc0 + jnp.abs(ov[r, pl.ds(k * 16, 16)] - gv[r, pl.ds(k * 16, 16)])
            acc1 = acc1 + jnp.abs(ov[r, pl.ds(k * 16 + 16, 16)] - gv[r, pl.ds(k * 16 + 16, 16)])
        cs = plsc.cumsum(acc0 + acc1)
        plsc.store_scatter(
            out_v, [jnp.full((16,), r_out, jnp.int32)], jnp.exp(-cs), mask=last_lane
        )

    pltpu.async_copy(x_hbm.at[pl.ds(stripe, CH)], own_v[0], osem[0])
    h_idx.wait()
    pltpu.async_copy(x_hbm.at[idx_v.at[pl.ds(0, CH)]], gth_v[0], gsem[0])
    for b in range(1, NBUF):
        issue(b, b)

    def superchunk(s, carry):
        for bi in range(NBUF):
            ci = s * NBUF + bi
            off = ci * CH
            pltpu.make_async_copy(x_hbm.at[pl.ds(0, CH)], gth_v[bi], gsem[bi]).wait()
            pltpu.make_async_copy(x_hbm.at[pl.ds(0, CH)], own_v[bi], osem[bi]).wait()

            ov, gv = own_v[bi], gth_v[bi]

            @plsc.parallel_loop(0, CH, step=1, unroll=RUNROLL)
            def _rows(r):
                compute_row(r, off + r, ov, gv)

            pltpu.async_copy(
                out_v.at[pl.ds(off, CH)], out_hbm.at[pl.ds(stripe + off, CH)], wsem
            )

            @pl.when(ci + NBUF < NCHUNK)
            def _prefetch():
                issue(ci + NBUF, bi)

        return carry

    lax.fori_loop(0, NCHUNK // NBUF, superchunk, 0)

    pltpu.make_async_copy(out_v, out_hbm.at[pl.ds(stripe, BPW)], wsem).wait()


def kernel(x, idx):
    return _sim_kernel(x, idx).reshape(B, 1)
